# baseline (speedup 1.0000x reference)
import functools
import math

import jax
import jax.numpy as jnp
from jax import lax
from jax.experimental import pallas as pl
from jax.experimental.pallas import tpu as pltpu
from jax.experimental.pallas import tpu_sc as plsc

_B = 16384
_RV = 512
_AV = 49
_AVP = 64
_ED = 32
_NA = 6
_NC = 2
_NS = 16
_NW = _NC * _NS
_SPT = _B // _NW
_SV = _SPT // 16

_HALF_PI = math.pi / 2.0
_PI = math.pi
_RAD2DEG = 180.0 / math.pi
_LN2 = float(math.log(2.0))
_SQRT2 = float(math.sqrt(2.0))

_ATAN_C = (
    9.9999990555e-01, -3.3332657853e-01, 1.9986537489e-01, -1.4164333375e-01,
    1.0507319787e-01, -7.2479506624e-02, 3.9899560043e-02, -1.4458697070e-02,
    2.4682466253e-03,
)
_LOG_C = (
    9.9999984747e-01, -5.0000696904e-01, 3.3335688093e-01, -2.4957780629e-01,
    1.9885361051e-01, -1.7363152975e-01, 1.6338556175e-01, -9.9136561212e-02,
)


def _sc_body(ap_hbm, gp_hbm, rt_hbm, at_hbm, w_hbm, b_hbm, out_hbm,
             ap_v, gp_v, rt_v, at_v, w_v, b_v, chunk_v,
             pr_sh, pa_sh, pr_v, pa_v, out_v):
    c = lax.axis_index("c")
    s = lax.axis_index("s")
    wid = s * _NC + c
    base = wid * _SPT
    iota = lax.broadcasted_iota(jnp.int32, (16,), 0)

    pltpu.sync_copy(ap_hbm.at[pl.ds(base, _SPT)], ap_v)
    pltpu.sync_copy(gp_hbm.at[pl.ds(base, _SPT)], gp_v)
    pltpu.sync_copy(rt_hbm.at[pl.ds(s * 32, 32)], rt_v)
    pltpu.sync_copy(w_hbm, w_v)
    pltpu.sync_copy(b_hbm, b_v)

    bvec = b_v[pl.ds(0, 16)]
    wr = [[w_v[a, pl.ds(ch * 16, 16)] for ch in range(2)] for a in range(_NA)]
    wa = [[w_v[a, pl.ds(_ED + ch * 16, 16)] for ch in range(2)]
          for a in range(_NA)]
    acc0 = [jnp.full((16,), bvec[a]) for a in range(_NA)]
    acc1 = [jnp.full((16,), bvec[a]) for a in range(_NA)]
    for d in range(_ED):
        dvec = jnp.full((16,), d, jnp.int32)
        col0 = plsc.load_gather(rt_v, [iota, dvec])
        col1 = plsc.load_gather(rt_v, [iota + 16, dvec])
        for a in range(_NA):
            w = wr[a][d // 16][d % 16]
            acc0[a] = acc0[a] + col0 * w
            acc1[a] = acc1[a] + col1 * w
    for a in range(_NA):
        chunk_v[a, pl.ds(0, 16)] = acc0[a]
        chunk_v[a, pl.ds(16, 16)] = acc1[a]
    for a in range(_NA):
        pltpu.sync_copy(chunk_v.at[a], pr_sh.at[a, pl.ds(s * 32, 32)])

    @pl.when(s < 4)
    def _angle_proj():
        pltpu.sync_copy(at_hbm.at[pl.ds(s * 16, 16)], at_v)
        aacc = [jnp.zeros((16,), jnp.float32) for _ in range(_NA)]
        for d in range(_ED):
            dvec = jnp.full((16,), d, jnp.int32)
            col = plsc.load_gather(at_v, [iota, dvec])
            for a in range(_NA):
                aacc[a] = aacc[a] + col * wa[a][d // 16][d % 16]
        for a in range(_NA):
            chunk_v[a, pl.ds(0, 16)] = aacc[a]
        for a in range(_NA):
            pltpu.sync_copy(chunk_v.at[a, pl.ds(0, 16)],
                            pa_sh.at[a, pl.ds(s * 16, 16)])

    plsc.subcore_barrier()
    pltpu.sync_copy(pr_sh, pr_v)
    pltpu.sync_copy(pa_sh, pa_v)

    col0i = jnp.full((16,), 0, jnp.int32)
    col1i = jnp.full((16,), 1, jnp.int32)
    col2i = jnp.full((16,), 2, jnp.int32)
    acols = [jnp.full((16,), a, jnp.int32) for a in range(_NA)]

    def body(i, carry):
        l = iota + i * 16
        ax = plsc.load_gather(ap_v, [l, col0i])
        az = plsc.load_gather(ap_v, [l, col1i])
        pose = plsc.load_gather(ap_v, [l, col2i])
        gx = plsc.load_gather(gp_v, [l, col0i])
        gz = plsc.load_gather(gp_v, [l, col1i])
        dx = gx - ax
        dz = gz - az
        d2 = dx * dx + dz * dz

        sb = lax.bitcast_convert_type(d2, jnp.int32)
        x = lax.bitcast_convert_type((sb >> 1) + 0x1FBD1DF5, jnp.float32)
        x = (x + d2 / x) * 0.5
        x = (x + d2 / x) * 0.5
        x = (x + d2 / x) * 0.5
        r_idx = (x / 5.0).astype(jnp.int32)

        axa = jnp.abs(dx)
        aya = jnp.abs(dz)
        swap = aya > axa
        num = jnp.where(swap, axa, aya)
        den = jnp.where(swap, aya, axa)
        t = num / den
        t = jnp.where(den == 0.0, 0.0, t)
        u = t * t
        p = jnp.float32(_ATAN_C[8])
        for k in range(7, -1, -1):
            p = p * u + _ATAN_C[k]
        p = p * t
        r = jnp.where(swap, _HALF_PI - p, p)
        r = jnp.where(dx < 0.0, _PI - r, r)
        r = jnp.where(dz < 0.0, -r, r)

        ang = 90.0 - r * _RAD2DEG
        diff = ang - pose
        rm = lax.rem(diff, jnp.float32(360.0))
        m = jnp.where(rm < 0.0, rm + 360.0, rm)
        t_idx = (m / 7.5).astype(jnp.int32)

        logits = []
        for a in range(_NA):
            lr = plsc.load_gather(pr_v, [acols[a], r_idx])
            la = plsc.load_gather(pa_v, [acols[a], t_idx])
            logits.append(lr + la)
        mx = logits[0]
        for a in range(1, _NA):
            mx = jnp.maximum(mx, logits[a])
        sh = [v - mx for v in logits]
        es = [jnp.exp(v) for v in sh]
        tot = ((es[0] + es[1]) + (es[2] + es[3])) + (es[4] + es[5])

        tb = lax.bitcast_convert_type(tot, jnp.int32)
        e = (tb >> 23) - 127
        mb = (tb & 0x7FFFFF) | (127 << 23)
        mf = lax.bitcast_convert_type(mb, jnp.float32)
        big = mf > _SQRT2
        mf = jnp.where(big, mf * 0.5, mf)
        e = e + big.astype(jnp.int32)
        z = mf - 1.0
        q = jnp.float32(_LOG_C[7])
        for k in range(6, -1, -1):
            q = q * z + _LOG_C[k]
        lse = e.astype(jnp.float32) * _LN2 + q * z

        for a in range(_NA):
            plsc.store_scatter(out_v, [l, acols[a]], sh[a] - lse)
        return carry

    lax.fori_loop(0, _SV, body, 0)
    pltpu.sync_copy(out_v, out_hbm.at[pl.ds(base, _SPT)])


@functools.partial(
    pl.kernel,
    out_type=jax.ShapeDtypeStruct((_B, _NA), jnp.float32),
    mesh=plsc.VectorSubcoreMesh(core_axis_name="c", subcore_axis_name="s",
                                num_cores=_NC, num_subcores=_NS),
    compiler_params=pltpu.CompilerParams(needs_layout_passes=False,
                                         use_tc_tiling_on_sc=False),
    scratch_types=[
        pltpu.VMEM((_SPT, 3), jnp.float32),
        pltpu.VMEM((_SPT, 2), jnp.float32),
        pltpu.VMEM((32, _ED), jnp.float32),
        pltpu.VMEM((16, _ED), jnp.float32),
        pltpu.VMEM((_NA, 2 * _ED), jnp.float32),
        pltpu.VMEM((16,), jnp.float32),
        pltpu.VMEM((_NA, 32), jnp.float32),
        pltpu.VMEM_SHARED((_NA, _RV), jnp.float32),
        pltpu.VMEM_SHARED((_NA, _AVP), jnp.float32),
        pltpu.VMEM((_NA, _RV), jnp.float32),
        pltpu.VMEM((_NA, _AVP), jnp.float32),
        pltpu.VMEM((_SPT, _NA), jnp.float32),
    ],
)
def _sc_kernel(ap_hbm, gp_hbm, rt_hbm, at_hbm, w_hbm, b_hbm, out_hbm,
               ap_v, gp_v, rt_v, at_v, w_v, b_v, chunk_v,
               pr_sh, pa_sh, pr_v, pa_v, out_v):
    _sc_body(ap_hbm, gp_hbm, rt_hbm, at_hbm, w_hbm, b_hbm, out_hbm,
             ap_v, gp_v, rt_v, at_v, w_v, b_v, chunk_v,
             pr_sh, pa_sh, pr_v, pa_v, out_v)


def kernel(agent_positions, goal_positions, radius_table, angle_table, W, b):
    at = jnp.pad(angle_table, ((0, _AVP - _AV), (0, 0)))
    bp = jnp.pad(b, (0, 16 - _NA))
    return _sc_kernel(agent_positions, goal_positions, radius_table, at, W, bp)


# speedup vs baseline: 2.3446x; 2.3446x over previous
import functools
import math

import jax
import jax.numpy as jnp
from jax import lax
from jax.experimental import pallas as pl
from jax.experimental.pallas import tpu as pltpu
from jax.experimental.pallas import tpu_sc as plsc

_B = 16384
_RV = 512
_AV = 49
_AVP = 64
_ED = 32
_NA = 6
_NC = 2
_NS = 16
_NW = _NC * _NS
_SPT = _B // _NW
_SV = _SPT // 16

_HALF_PI = math.pi / 2.0
_PI = math.pi
_RAD2DEG = 180.0 / math.pi
_LN2 = float(math.log(2.0))
_SQRT2 = float(math.sqrt(2.0))

_ATAN_C = (
    9.9999990555e-01, -3.3332657853e-01, 1.9986537489e-01, -1.4164333375e-01,
    1.0507319787e-01, -7.2479506624e-02, 3.9899560043e-02, -1.4458697070e-02,
    2.4682466253e-03,
)
_LOG_C = (
    9.9999984747e-01, -5.0000696904e-01, 3.3335688093e-01, -2.4957780629e-01,
    1.9885361051e-01, -1.7363152975e-01, 1.6338556175e-01, -9.9136561212e-02,
)


def _sc_body(ap_hbm, gp_hbm, rt_hbm, at_hbm, w_hbm, b_hbm, out_hbm,
             ap_v, gp_v, rt_v, at_v, w_v, b_v, chunk_v,
             pr_sh, pa_sh, pr_v, pa_v, out_v):
    c = lax.axis_index("c")
    s = lax.axis_index("s")
    wid = s * _NC + c
    base = wid * _SPT
    iota = lax.broadcasted_iota(jnp.int32, (16,), 0)

    pltpu.sync_copy(ap_hbm.at[:, pl.ds(base, _SPT)], ap_v)
    pltpu.sync_copy(gp_hbm.at[:, pl.ds(base, _SPT)], gp_v)
    pltpu.sync_copy(rt_hbm.at[pl.ds(s * 32, 32)], rt_v)
    pltpu.sync_copy(w_hbm, w_v)
    pltpu.sync_copy(b_hbm, b_v)

    bvec = b_v[pl.ds(0, 16)]
    wr = [[w_v[a, pl.ds(ch * 16, 16)] for ch in range(2)] for a in range(_NA)]
    wa = [[w_v[a, pl.ds(_ED + ch * 16, 16)] for ch in range(2)]
          for a in range(_NA)]
    acc0 = [jnp.full((16,), bvec[a]) for a in range(_NA)]
    acc1 = [jnp.full((16,), bvec[a]) for a in range(_NA)]
    for d in range(_ED):
        dvec = jnp.full((16,), d, jnp.int32)
        col0 = plsc.load_gather(rt_v, [iota, dvec])
        col1 = plsc.load_gather(rt_v, [iota + 16, dvec])
        for a in range(_NA):
            w = wr[a][d // 16][d % 16]
            acc0[a] = acc0[a] + col0 * w
            acc1[a] = acc1[a] + col1 * w
    for a in range(_NA):
        chunk_v[a, pl.ds(0, 16)] = acc0[a]
        chunk_v[a, pl.ds(16, 16)] = acc1[a]
    for a in range(_NA):
        pltpu.sync_copy(chunk_v.at[a], pr_sh.at[a, pl.ds(s * 32, 32)])

    @pl.when(s < 4)
    def _angle_proj():
        pltpu.sync_copy(at_hbm.at[pl.ds(s * 16, 16)], at_v)
        aacc = [jnp.zeros((16,), jnp.float32) for _ in range(_NA)]
        for d in range(_ED):
            dvec = jnp.full((16,), d, jnp.int32)
            col = plsc.load_gather(at_v, [iota, dvec])
            for a in range(_NA):
                aacc[a] = aacc[a] + col * wa[a][d // 16][d % 16]
        for a in range(_NA):
            chunk_v[a, pl.ds(0, 16)] = aacc[a]
        for a in range(_NA):
            pltpu.sync_copy(chunk_v.at[a, pl.ds(0, 16)],
                            pa_sh.at[a, pl.ds(s * 16, 16)])

    plsc.subcore_barrier()
    pltpu.sync_copy(pr_sh, pr_v)
    pltpu.sync_copy(pa_sh, pa_v)

    acols = [jnp.full((16,), a, jnp.int32) for a in range(_NA)]

    def body(i, carry):
        o = i * 16
        ax = ap_v[0, pl.ds(o, 16)]
        az = ap_v[1, pl.ds(o, 16)]
        pose = ap_v[2, pl.ds(o, 16)]
        gx = gp_v[0, pl.ds(o, 16)]
        gz = gp_v[1, pl.ds(o, 16)]
        dx = gx - ax
        dz = gz - az
        d2 = dx * dx + dz * dz

        sb = lax.bitcast_convert_type(d2, jnp.int32)
        x = lax.bitcast_convert_type((sb >> 1) + 0x1FBD1DF5, jnp.float32)
        x = (x + d2 / x) * 0.5
        x = (x + d2 / x) * 0.5
        x = (x + d2 / x) * 0.5
        r_idx = (x / 5.0).astype(jnp.int32)

        axa = jnp.abs(dx)
        aya = jnp.abs(dz)
        swap = aya > axa
        num = jnp.where(swap, axa, aya)
        den = jnp.where(swap, aya, axa)
        t = num / den
        t = jnp.where(den == 0.0, 0.0, t)
        u = t * t
        p = jnp.float32(_ATAN_C[8])
        for k in range(7, -1, -1):
            p = p * u + _ATAN_C[k]
        p = p * t
        r = jnp.where(swap, _HALF_PI - p, p)
        r = jnp.where(dx < 0.0, _PI - r, r)
        r = jnp.where(dz < 0.0, -r, r)

        ang = 90.0 - r * _RAD2DEG
        diff = ang - pose
        rm = lax.rem(diff, jnp.float32(360.0))
        m = jnp.where(rm < 0.0, rm + 360.0, rm)
        t_idx = (m / 7.5).astype(jnp.int32)

        logits = []
        for a in range(_NA):
            lr = plsc.load_gather(pr_v, [acols[a], r_idx])
            la = plsc.load_gather(pa_v, [acols[a], t_idx])
            logits.append(lr + la)
        mx = logits[0]
        for a in range(1, _NA):
            mx = jnp.maximum(mx, logits[a])
        sh = [v - mx for v in logits]
        es = [jnp.exp(v) for v in sh]
        tot = ((es[0] + es[1]) + (es[2] + es[3])) + (es[4] + es[5])

        tb = lax.bitcast_convert_type(tot, jnp.int32)
        e = (tb >> 23) - 127
        mb = (tb & 0x7FFFFF) | (127 << 23)
        mf = lax.bitcast_convert_type(mb, jnp.float32)
        big = mf > _SQRT2
        mf = jnp.where(big, mf * 0.5, mf)
        e = e + big.astype(jnp.int32)
        z = mf - 1.0
        q = jnp.float32(_LOG_C[7])
        for k in range(6, -1, -1):
            q = q * z + _LOG_C[k]
        lse = e.astype(jnp.float32) * _LN2 + q * z

        for a in range(_NA):
            out_v[a, pl.ds(o, 16)] = sh[a] - lse
        return carry

    lax.fori_loop(0, _SV, body, 0)
    pltpu.sync_copy(out_v, out_hbm.at[:, pl.ds(base, _SPT)])


@functools.partial(
    pl.kernel,
    out_type=jax.ShapeDtypeStruct((_NA, _B), jnp.float32),
    mesh=plsc.VectorSubcoreMesh(core_axis_name="c", subcore_axis_name="s",
                                num_cores=_NC, num_subcores=_NS),
    compiler_params=pltpu.CompilerParams(needs_layout_passes=False,
                                         use_tc_tiling_on_sc=False),
    scratch_types=[
        pltpu.VMEM((3, _SPT), jnp.float32),
        pltpu.VMEM((2, _SPT), jnp.float32),
        pltpu.VMEM((32, _ED), jnp.float32),
        pltpu.VMEM((16, _ED), jnp.float32),
        pltpu.VMEM((_NA, 2 * _ED), jnp.float32),
        pltpu.VMEM((16,), jnp.float32),
        pltpu.VMEM((_NA, 32), jnp.float32),
        pltpu.VMEM_SHARED((_NA, _RV), jnp.float32),
        pltpu.VMEM_SHARED((_NA, _AVP), jnp.float32),
        pltpu.VMEM((_NA, _RV), jnp.float32),
        pltpu.VMEM((_NA, _AVP), jnp.float32),
        pltpu.VMEM((_NA, _SPT), jnp.float32),
    ],
)
def _sc_kernel(ap_hbm, gp_hbm, rt_hbm, at_hbm, w_hbm, b_hbm, out_hbm,
               ap_v, gp_v, rt_v, at_v, w_v, b_v, chunk_v,
               pr_sh, pa_sh, pr_v, pa_v, out_v):
    _sc_body(ap_hbm, gp_hbm, rt_hbm, at_hbm, w_hbm, b_hbm, out_hbm,
             ap_v, gp_v, rt_v, at_v, w_v, b_v, chunk_v,
             pr_sh, pa_sh, pr_v, pa_v, out_v)


def kernel(agent_positions, goal_positions, radius_table, angle_table, W, b):
    at = jnp.pad(angle_table, ((0, _AVP - _AV), (0, 0)))
    bp = jnp.pad(b, (0, 16 - _NA))
    out = _sc_kernel(agent_positions.T, goal_positions.T, radius_table, at,
                     W, bp)
    return out.T
